# baseline (speedup 1.0000x reference)
import functools

import jax
import jax.numpy as jnp
from jax import lax
from jax.experimental import pallas as pl
from jax.experimental.pallas import tpu as pltpu
from jax.experimental.pallas import tpu_sc as plsc

N_UNITS = 100000
N_BATCH = 1024

NC = 2
NS = 16
NW = NC * NS

B_PAD = 102400
PW = B_PAD // NW
CH = 40
NCHUNK = PW // CH

_mesh = plsc.VectorSubcoreMesh(
    core_axis_name="c", subcore_axis_name="s", num_cores=NC, num_subcores=NS
)


@functools.partial(
    pl.kernel,
    out_type=jax.ShapeDtypeStruct((B_PAD, N_BATCH), jnp.float32),
    mesh=_mesh,
    scratch_types=[
        pltpu.VMEM((PW,), jnp.int32),
        pltpu.VMEM((CH, N_BATCH), jnp.float32),
        pltpu.VMEM((CH, N_BATCH), jnp.float32),
        pltpu.SemaphoreType.DMA,
        pltpu.SemaphoreType.DMA,
        pltpu.SemaphoreType.DMA,
        pltpu.SemaphoreType.DMA,
    ],
)
def _sc_row_gather(tab_hbm, idx_hbm, out_hbm, idx_v, buf0, buf1, gs0, gs1, ws0, ws1):
    wid = lax.axis_index("s") * NC + lax.axis_index("c")
    base = pl.multiple_of(wid * PW, 8)
    pltpu.sync_copy(idx_hbm.at[pl.ds(base, PW)], idx_v)

    bufs = (buf0, buf1)
    gsems = (gs0, gs1)
    wsems = (ws0, ws1)

    def start_gather(c, b):
        off = pl.multiple_of(c * CH, 8)
        pltpu.async_copy(tab_hbm.at[idx_v.at[pl.ds(off, CH)]], bufs[b], gsems[b])

    def wait_gather(b):
        pltpu.make_async_copy(
            tab_hbm.at[idx_v.at[pl.ds(0, CH)]], bufs[b], gsems[b]
        ).wait()

    start_gather(0, 0)
    start_gather(1, 1)

    def step(g, carry):
        for b in range(2):
            c = g + b
            wait_gather(b)
            off = pl.multiple_of(c * CH, 8)
            pltpu.async_copy(bufs[b], out_hbm.at[pl.ds(base + off, CH)], wsems[b])
            pltpu.make_async_copy(
                bufs[b], out_hbm.at[pl.ds(base, CH)], wsems[b]
            ).wait()

            @pl.when(c + 2 < NCHUNK)
            def _():
                start_gather(c + 2, b)

        return carry

    lax.fori_loop(0, NCHUNK // 2, lambda i, car: step(i * 2, car), 0)


def kernel(z, P):
    zT = z.T
    pad = jnp.arange(B_PAD - N_UNITS, dtype=jnp.int32)
    P_pad = jnp.concatenate([P.astype(jnp.int32), pad])
    outT = _sc_row_gather(zT, P_pad)
    return outT[:N_UNITS].T


# speedup vs baseline: 1.8699x; 1.8699x over previous
import functools

import jax
import jax.numpy as jnp
from jax import lax
from jax.experimental import pallas as pl
from jax.experimental.pallas import tpu as pltpu
from jax.experimental.pallas import tpu_sc as plsc

N_UNITS = 100000
N_BATCH = 1024

NC = 2
NS = 16
NW = NC * NS

CH = 40
NCHUNK = 78
BIG = 20
TAIL = 8
PW_MAX = NCHUNK * CH + TAIL

_mesh = plsc.VectorSubcoreMesh(
    core_axis_name="c", subcore_axis_name="s", num_cores=NC, num_subcores=NS
)


@functools.partial(
    pl.kernel,
    out_type=jax.ShapeDtypeStruct((N_UNITS, N_BATCH), jnp.float32),
    mesh=_mesh,
    scratch_types=[
        pltpu.VMEM((PW_MAX,), jnp.int32),
        pltpu.VMEM((CH, N_BATCH), jnp.float32),
        pltpu.VMEM((CH, N_BATCH), jnp.float32),
        pltpu.SemaphoreType.DMA,
        pltpu.SemaphoreType.DMA,
        pltpu.SemaphoreType.DMA,
        pltpu.SemaphoreType.DMA,
    ],
)
def _sc_row_gather(tab_hbm, idx_hbm, out_hbm, idx_v, buf0, buf1, gs0, gs1, ws0, ws1):
    wid = lax.axis_index("s") * NC + lax.axis_index("c")
    base = pl.multiple_of(wid * (NCHUNK * CH) + jnp.minimum(wid, BIG) * TAIL, 8)
    big = wid < BIG

    pltpu.sync_copy(idx_hbm.at[pl.ds(base, NCHUNK * CH)],
                    idx_v.at[pl.ds(0, NCHUNK * CH)])

    bufs = (buf0, buf1)
    gsems = (gs0, gs1)
    wsems = (ws0, ws1)

    def start_gather(c, b):
        off = pl.multiple_of(c * CH, 8)
        pltpu.async_copy(tab_hbm.at[idx_v.at[pl.ds(off, CH)]], bufs[b], gsems[b])

    def wait_gather(b):
        pltpu.make_async_copy(
            tab_hbm.at[idx_v.at[pl.ds(0, CH)]], bufs[b], gsems[b]
        ).wait()

    start_gather(0, 0)
    start_gather(1, 1)

    def step(g, carry):
        for b in range(2):
            c = g + b
            wait_gather(b)
            off = pl.multiple_of(c * CH, 8)
            pltpu.async_copy(bufs[b], out_hbm.at[pl.ds(base + off, CH)], wsems[b])
            pltpu.make_async_copy(
                bufs[b], out_hbm.at[pl.ds(base, CH)], wsems[b]
            ).wait()

            @pl.when(c + 2 < NCHUNK)
            def _():
                start_gather(c + 2, b)

        return carry

    lax.fori_loop(0, NCHUNK // 2, lambda i, car: step(i * 2, car), 0)

    @pl.when(big)
    def _():
        toff = pl.multiple_of(NCHUNK * CH, 8)
        pltpu.sync_copy(idx_hbm.at[pl.ds(base + toff, TAIL)],
                        idx_v.at[pl.ds(toff, TAIL)])
        tbuf = buf0.at[pl.ds(0, TAIL)]
        pltpu.async_copy(tab_hbm.at[idx_v.at[pl.ds(toff, TAIL)]], tbuf, gs0)
        pltpu.make_async_copy(
            tab_hbm.at[idx_v.at[pl.ds(toff, TAIL)]], tbuf, gs0
        ).wait()
        pltpu.async_copy(tbuf, out_hbm.at[pl.ds(base + toff, TAIL)], ws0)
        pltpu.make_async_copy(tbuf, out_hbm.at[pl.ds(base, TAIL)], ws0).wait()


def kernel(z, P):
    zT = z.T
    outT = _sc_row_gather(zT, P.astype(jnp.int32))
    return outT.T


# speedup vs baseline: 1.8712x; 1.0007x over previous
import functools

import jax
import jax.numpy as jnp
from jax import lax
from jax.experimental import pallas as pl
from jax.experimental.pallas import tpu as pltpu
from jax.experimental.pallas import tpu_sc as plsc

N_UNITS = 100000
N_BATCH = 1024

NC = 2
NS = 16
NW = NC * NS

CH = 40
NCHUNK = 78
BIG = 20
TAIL = 8
PW_MAX = NCHUNK * CH + TAIL

_mesh = plsc.VectorSubcoreMesh(
    core_axis_name="c", subcore_axis_name="s", num_cores=NC, num_subcores=NS
)


@functools.partial(
    pl.kernel,
    out_type=jax.ShapeDtypeStruct((N_UNITS, N_BATCH), jnp.float32),
    mesh=_mesh,
    scratch_types=[
        pltpu.VMEM((PW_MAX,), jnp.int32),
        pltpu.VMEM((CH, N_BATCH), jnp.float32),
        pltpu.VMEM((CH, N_BATCH), jnp.float32),
        pltpu.VMEM((CH, N_BATCH), jnp.float32),
        pltpu.SemaphoreType.DMA,
        pltpu.SemaphoreType.DMA,
        pltpu.SemaphoreType.DMA,
        pltpu.SemaphoreType.DMA,
        pltpu.SemaphoreType.DMA,
        pltpu.SemaphoreType.DMA,
    ],
)
def _sc_row_gather(tab_hbm, idx_hbm, out_hbm, idx_v, buf0, buf1, buf2,
                   gs0, gs1, gs2, ws0, ws1, ws2):
    wid = lax.axis_index("s") * NC + lax.axis_index("c")
    base = pl.multiple_of(wid * (NCHUNK * CH) + jnp.minimum(wid, BIG) * TAIL, 8)
    big = wid < BIG

    pltpu.sync_copy(idx_hbm.at[pl.ds(base, NCHUNK * CH)],
                    idx_v.at[pl.ds(0, NCHUNK * CH)])

    bufs = (buf0, buf1, buf2)
    gsems = (gs0, gs1, gs2)
    wsems = (ws0, ws1, ws2)

    def start_gather(c, b):
        off = pl.multiple_of(c * CH, 8)
        pltpu.async_copy(tab_hbm.at[idx_v.at[pl.ds(off, CH)]], bufs[b], gsems[b])

    def wait_gather(b):
        pltpu.make_async_copy(
            tab_hbm.at[idx_v.at[pl.ds(0, CH)]], bufs[b], gsems[b]
        ).wait()

    start_gather(0, 0)
    start_gather(1, 1)
    start_gather(2, 2)

    def step(g, carry):
        for b in range(3):
            c = g + b
            wait_gather(b)
            off = pl.multiple_of(c * CH, 8)
            pltpu.async_copy(bufs[b], out_hbm.at[pl.ds(base + off, CH)], wsems[b])
            pltpu.make_async_copy(
                bufs[b], out_hbm.at[pl.ds(base, CH)], wsems[b]
            ).wait()

            @pl.when(c + 3 < NCHUNK)
            def _():
                start_gather(c + 3, b)

        return carry

    lax.fori_loop(0, NCHUNK // 3, lambda i, car: step(i * 3, car), 0)

    @pl.when(big)
    def _():
        toff = pl.multiple_of(NCHUNK * CH, 8)
        pltpu.sync_copy(idx_hbm.at[pl.ds(base + toff, TAIL)],
                        idx_v.at[pl.ds(toff, TAIL)])
        tbuf = buf0.at[pl.ds(0, TAIL)]
        pltpu.async_copy(tab_hbm.at[idx_v.at[pl.ds(toff, TAIL)]], tbuf, gs0)
        pltpu.make_async_copy(
            tab_hbm.at[idx_v.at[pl.ds(toff, TAIL)]], tbuf, gs0
        ).wait()
        pltpu.async_copy(tbuf, out_hbm.at[pl.ds(base + toff, TAIL)], ws0)
        pltpu.make_async_copy(tbuf, out_hbm.at[pl.ds(base, TAIL)], ws0).wait()


def kernel(z, P):
    zT = z.T
    outT = _sc_row_gather(zT, P.astype(jnp.int32))
    return outT.T


# speedup vs baseline: 1.8737x; 1.0013x over previous
import functools

import jax
import jax.numpy as jnp
from jax import lax
from jax.experimental import pallas as pl
from jax.experimental.pallas import tpu as pltpu
from jax.experimental.pallas import tpu_sc as plsc

N_UNITS = 100000
N_BATCH = 1024

NC = 2
NS = 16
NW = NC * NS

CH = 24
NCHUNK = 130
BIG = 20
TAIL = 8
PW_MAX = NCHUNK * CH + TAIL

_mesh = plsc.VectorSubcoreMesh(
    core_axis_name="c", subcore_axis_name="s", num_cores=NC, num_subcores=NS
)


@functools.partial(
    pl.kernel,
    out_type=jax.ShapeDtypeStruct((N_UNITS, N_BATCH), jnp.float32),
    mesh=_mesh,
    scratch_types=[
        pltpu.VMEM((PW_MAX,), jnp.int32),
        pltpu.VMEM((CH, N_BATCH), jnp.float32),
        pltpu.VMEM((CH, N_BATCH), jnp.float32),
        pltpu.VMEM((CH, N_BATCH), jnp.float32),
        pltpu.VMEM((CH, N_BATCH), jnp.float32),
        pltpu.VMEM((CH, N_BATCH), jnp.float32),
        pltpu.SemaphoreType.DMA,
        pltpu.SemaphoreType.DMA,
        pltpu.SemaphoreType.DMA,
        pltpu.SemaphoreType.DMA,
        pltpu.SemaphoreType.DMA,
        pltpu.SemaphoreType.DMA,
        pltpu.SemaphoreType.DMA,
        pltpu.SemaphoreType.DMA,
        pltpu.SemaphoreType.DMA,
        pltpu.SemaphoreType.DMA,
    ],
)
def _sc_row_gather(tab_hbm, idx_hbm, out_hbm, idx_v, buf0, buf1, buf2, buf3, buf4,
                   gs0, gs1, gs2, gs3, gs4, ws0, ws1, ws2, ws3, ws4):
    wid = lax.axis_index("s") * NC + lax.axis_index("c")
    base = pl.multiple_of(wid * (NCHUNK * CH) + jnp.minimum(wid, BIG) * TAIL, 8)
    big = wid < BIG

    pltpu.sync_copy(idx_hbm.at[pl.ds(base, NCHUNK * CH)],
                    idx_v.at[pl.ds(0, NCHUNK * CH)])

    bufs = (buf0, buf1, buf2, buf3, buf4)
    gsems = (gs0, gs1, gs2, gs3, gs4)
    wsems = (ws0, ws1, ws2, ws3, ws4)

    def start_gather(c, b):
        off = pl.multiple_of(c * CH, 8)
        pltpu.async_copy(tab_hbm.at[idx_v.at[pl.ds(off, CH)]], bufs[b], gsems[b])

    def wait_gather(b):
        pltpu.make_async_copy(
            tab_hbm.at[idx_v.at[pl.ds(0, CH)]], bufs[b], gsems[b]
        ).wait()

    for b in range(5):
        start_gather(b, b)

    def step(g, carry):
        for b in range(5):
            c = g + b
            wait_gather(b)
            off = pl.multiple_of(c * CH, 8)
            pltpu.async_copy(bufs[b], out_hbm.at[pl.ds(base + off, CH)], wsems[b])
            pltpu.make_async_copy(
                bufs[b], out_hbm.at[pl.ds(base, CH)], wsems[b]
            ).wait()

            @pl.when(c + 5 < NCHUNK)
            def _():
                start_gather(c + 5, b)

        return carry

    lax.fori_loop(0, NCHUNK // 5, lambda i, car: step(i * 5, car), 0)

    @pl.when(big)
    def _():
        toff = pl.multiple_of(NCHUNK * CH, 8)
        pltpu.sync_copy(idx_hbm.at[pl.ds(base + toff, TAIL)],
                        idx_v.at[pl.ds(toff, TAIL)])
        tbuf = buf0.at[pl.ds(0, TAIL)]
        pltpu.async_copy(tab_hbm.at[idx_v.at[pl.ds(toff, TAIL)]], tbuf, gs0)
        pltpu.make_async_copy(
            tab_hbm.at[idx_v.at[pl.ds(toff, TAIL)]], tbuf, gs0
        ).wait()
        pltpu.async_copy(tbuf, out_hbm.at[pl.ds(base + toff, TAIL)], ws0)
        pltpu.make_async_copy(tbuf, out_hbm.at[pl.ds(base, TAIL)], ws0).wait()


def kernel(z, P):
    zT = z.T
    outT = _sc_row_gather(zT, P.astype(jnp.int32))
    return outT.T
